# baseline (speedup 1.0000x reference)
import jax
import jax.numpy as jnp
from jax.experimental import pallas as pl
from jax.experimental.pallas import tpu as pltpu

TOKENS = 16384
D_MODEL = 2048
NUM_EXPERTS = 16
BLOCK_T = 1024


def _gating_kernel(x_ref, wt_ref, b_ref, w_ref, idx_ref, topw_ref):
    s = jnp.dot(x_ref[...], wt_ref[...], preferred_element_type=jnp.float32)
    s = s + b_ref[...]
    iota = jax.lax.broadcasted_iota(jnp.int32, s.shape, 1)

    i0 = jnp.argmax(s, axis=1).astype(jnp.int32)
    m0 = jnp.max(s, axis=1)
    masked = jnp.where(iota == i0[:, None], -jnp.inf, s)
    i1 = jnp.argmax(masked, axis=1).astype(jnp.int32)
    m1 = jnp.max(masked, axis=1)

    e = jnp.exp(m1 - m0)
    w0 = 1.0 / (1.0 + e)
    w1 = e * w0

    w_ref[...] = jnp.where(
        iota == i0[:, None], w0[:, None],
        jnp.where(iota == i1[:, None], w1[:, None], 0.0))
    idx_ref[...] = jnp.concatenate([i0[:, None], i1[:, None]], axis=1)
    topw_ref[...] = jnp.concatenate([w0[:, None], w1[:, None]], axis=1)


def kernel(x, gate_w, gate_b):
    wt = gate_w.T
    b2 = gate_b.reshape(1, NUM_EXPERTS)
    grid = (TOKENS // BLOCK_T,)
    weights, idx, topw = pl.pallas_call(
        _gating_kernel,
        grid=grid,
        in_specs=[
            pl.BlockSpec((BLOCK_T, D_MODEL), lambda i: (i, 0)),
            pl.BlockSpec((D_MODEL, NUM_EXPERTS), lambda i: (0, 0)),
            pl.BlockSpec((1, NUM_EXPERTS), lambda i: (0, 0)),
        ],
        out_specs=[
            pl.BlockSpec((BLOCK_T, NUM_EXPERTS), lambda i: (i, 0)),
            pl.BlockSpec((BLOCK_T, 2), lambda i: (i, 0)),
            pl.BlockSpec((BLOCK_T, 2), lambda i: (i, 0)),
        ],
        out_shape=[
            jax.ShapeDtypeStruct((TOKENS, NUM_EXPERTS), jnp.float32),
            jax.ShapeDtypeStruct((TOKENS, 2), jnp.int32),
            jax.ShapeDtypeStruct((TOKENS, 2), jnp.float32),
        ],
        compiler_params=pltpu.CompilerParams(
            dimension_semantics=("parallel",)),
    )(x, wt, b2)
    return (weights, idx, topw)


# speedup vs baseline: 1.0092x; 1.0092x over previous
import jax
import jax.numpy as jnp
from jax.experimental import pallas as pl
from jax.experimental.pallas import tpu as pltpu

TOKENS = 16384
D_MODEL = 2048
NUM_EXPERTS = 16
BLOCK_T = 2048


def _gating_kernel(x_ref, wt_ref, b_ref, w_ref, idx_ref, topw_ref):
    s = jnp.dot(x_ref[...], wt_ref[...], preferred_element_type=jnp.float32)
    s = s + b_ref[...]
    iota = jax.lax.broadcasted_iota(jnp.int32, s.shape, 1)

    i0 = jnp.argmax(s, axis=1).astype(jnp.int32)
    m0 = jnp.max(s, axis=1)
    masked = jnp.where(iota == i0[:, None], -jnp.inf, s)
    i1 = jnp.argmax(masked, axis=1).astype(jnp.int32)
    m1 = jnp.max(masked, axis=1)

    e = jnp.exp(m1 - m0)
    w0 = 1.0 / (1.0 + e)
    w1 = e * w0

    w_ref[...] = jnp.where(
        iota == i0[:, None], w0[:, None],
        jnp.where(iota == i1[:, None], w1[:, None], 0.0))
    idx_ref[...] = jnp.concatenate([i0[:, None], i1[:, None]], axis=1)
    topw_ref[...] = jnp.concatenate([w0[:, None], w1[:, None]], axis=1)


def kernel(x, gate_w, gate_b):
    wt = gate_w.T
    b2 = gate_b.reshape(1, NUM_EXPERTS)
    grid = (TOKENS // BLOCK_T,)
    weights, idx, topw = pl.pallas_call(
        _gating_kernel,
        grid=grid,
        in_specs=[
            pl.BlockSpec((BLOCK_T, D_MODEL), lambda i: (i, 0)),
            pl.BlockSpec((D_MODEL, NUM_EXPERTS), lambda i: (0, 0)),
            pl.BlockSpec((1, NUM_EXPERTS), lambda i: (0, 0)),
        ],
        out_specs=[
            pl.BlockSpec((BLOCK_T, NUM_EXPERTS), lambda i: (i, 0)),
            pl.BlockSpec((BLOCK_T, 2), lambda i: (i, 0)),
            pl.BlockSpec((BLOCK_T, 2), lambda i: (i, 0)),
        ],
        out_shape=[
            jax.ShapeDtypeStruct((TOKENS, NUM_EXPERTS), jnp.float32),
            jax.ShapeDtypeStruct((TOKENS, 2), jnp.int32),
            jax.ShapeDtypeStruct((TOKENS, 2), jnp.float32),
        ],
        compiler_params=pltpu.CompilerParams(
            dimension_semantics=("parallel",)),
    )(x, wt, b2)
    return (weights, idx, topw)
